# baseline (speedup 1.0000x reference)
import functools

import jax
import jax.numpy as jnp
from jax import lax
from jax.experimental import pallas as pl
from jax.experimental.pallas import tpu as pltpu
from jax.experimental.pallas import tpu_sc as plsc

EMB = 32
B_ROWS = 4096
B_COLS = 200
NW = 32
T_TILES = 32
UNITS = B_COLS * T_TILES
U_PER_W = UNITS // NW
NBUF = 4


def _emb_body(idx_hbm, tab_hbm, out_hbm, idx_v, rows_v, block_v, sems):
    isems, gsems, ssems = sems
    nc = 2
    wid = lax.axis_index("s") * nc + lax.axis_index("c")
    u_base = wid * U_PER_W

    l_iota = lax.iota(jnp.int32, 16)

    def unit_jt(u):
        uu = u_base + u
        j = uu // T_TILES
        t = uu % T_TILES
        return j, t

    def fire_idx(u, b):
        j, t = unit_jt(u)
        pltpu.async_copy(idx_hbm.at[j // 8, t, j % 8], idx_v.at[b], isems[b])

    def wait_idx(b):
        pltpu.make_async_copy(idx_hbm.at[0, 0, 0], idx_v.at[b], isems[b]).wait()

    def fire_gather(b):
        pltpu.async_copy(tab_hbm.at[idx_v.at[b]], rows_v.at[b], gsems[b])

    def wait_gather(b):
        pltpu.make_async_copy(tab_hbm.at[idx_v.at[b]], rows_v.at[b], gsems[b]).wait()

    def fire_store(u, b):
        j, t = unit_jt(u)
        pltpu.async_copy(block_v.at[b], out_hbm.at[j, :, t], ssems[b])

    def wait_store(b):
        pltpu.make_async_copy(block_v.at[b], out_hbm.at[0, :, 0], ssems[b]).wait()

    row_ids = [l_iota + (16 * l0) for l0 in range(8)]

    def transpose(b):
        @plsc.parallel_loop(0, EMB, 1, unroll=8)
        def col_loop(k):
            col = jnp.full((16,), 1, jnp.int32) * k
            for l0 in range(8):
                v = plsc.load_gather(rows_v.at[b], [row_ids[l0], col])
                block_v[b, k // 8, k % 8, pl.ds(16 * l0, 16)] = v

    for b in range(NBUF):
        fire_idx(b, b)
    for b in range(2):
        wait_idx(b)
        fire_gather(b)

    def group(g, carry):
        for b in range(NBUF):
            u = NBUF * g + b
            wait_gather(b)

            @pl.when(u + NBUF < U_PER_W)
            def _():
                fire_idx(u + NBUF, b)

            @pl.when(u >= NBUF)
            def _():
                wait_store(b)

            transpose(b)
            fire_store(u, b)

            b2 = (b + 2) % NBUF

            @pl.when(u + 2 < U_PER_W)
            def _():
                wait_idx(b2)
                fire_gather(b2)

        return carry

    lax.fori_loop(0, U_PER_W // NBUF, group, 0)
    for b in range(NBUF):
        wait_store(b)


@functools.partial(
    pl.kernel,
    mesh=plsc.VectorSubcoreMesh(core_axis_name="c", subcore_axis_name="s"),
    out_type=jax.ShapeDtypeStruct((B_COLS, 4, T_TILES, 8, 128), jnp.float32),
    scratch_types=[
        pltpu.VMEM((NBUF, 128), jnp.int32),
        pltpu.VMEM((NBUF, 128, EMB), jnp.float32),
        pltpu.VMEM((NBUF, 4, 8, 128), jnp.float32),
        (pltpu.SemaphoreType.DMA,) * NBUF,
        (pltpu.SemaphoreType.DMA,) * NBUF,
        (pltpu.SemaphoreType.DMA,) * NBUF,
    ],
    compiler_params=pltpu.CompilerParams(
        use_tc_tiling_on_sc=False, needs_layout_passes=False
    ),
)
def _emb_lookup(idx_hbm, tab_hbm, out_hbm, idx_v, rows_v, block_v, isems, gsems, ssems):
    _emb_body(idx_hbm, tab_hbm, out_hbm, idx_v, rows_v, block_v, (isems, gsems, ssems))


def kernel(vocab_ids, node_embs_weight):
    ids = vocab_ids.astype(jnp.int32)
    idx_native = ids.T.reshape(25, 8, T_TILES, 128).transpose(0, 2, 1, 3)
    out5 = _emb_lookup(idx_native, node_embs_weight)
    return out5.transpose(2, 4, 0, 1, 3).reshape(B_ROWS, B_COLS, EMB)


# speedup vs baseline: 1.3640x; 1.3640x over previous
import functools

import jax
import jax.numpy as jnp
from jax import lax
from jax.experimental import pallas as pl
from jax.experimental.pallas import tpu as pltpu
from jax.experimental.pallas import tpu_sc as plsc

EMB = 32
B_ROWS = 4096
B_COLS = 200
NW = 32
T_TILES = 32
UNITS = B_COLS * T_TILES
U_PER_W = UNITS // NW
NBUF = 4


def _emb_body(idx_hbm, tab_hbm, out_hbm, idx_v, rows_v, block_v, sems):
    isems, gsems, ssems = sems
    nc = 2
    wid = lax.axis_index("s") * nc + lax.axis_index("c")
    u_base = wid * U_PER_W

    l_iota = lax.iota(jnp.int32, 16)

    def unit_jt(u):
        uu = u_base + u
        j = uu // T_TILES
        t = uu % T_TILES
        return j, t

    def fire_idx(u, b):
        j, t = unit_jt(u)
        pltpu.async_copy(idx_hbm.at[j // 8, t, j % 8], idx_v.at[b], isems[b])

    def wait_idx(b):
        pltpu.make_async_copy(idx_hbm.at[0, 0, 0], idx_v.at[b], isems[b]).wait()

    def fire_gather(b):
        pltpu.async_copy(tab_hbm.at[idx_v.at[b]], rows_v.at[b], gsems[b])

    def wait_gather(b):
        pltpu.make_async_copy(tab_hbm.at[idx_v.at[b]], rows_v.at[b], gsems[b]).wait()

    def fire_store(u, b):
        j, t = unit_jt(u)
        pltpu.async_copy(block_v.at[b, :, :, pl.ds(0, 128)], out_hbm.at[j, :, t], ssems[b])

    def wait_store(b):
        pltpu.make_async_copy(
            block_v.at[b, :, :, pl.ds(0, 128)], out_hbm.at[0, :, 0], ssems[b]
        ).wait()

    g0 = l_iota // 8
    s0 = l_iota % 8
    g1 = g0 + 2

    def transpose(b):
        @plsc.parallel_loop(0, 128, 1, unroll=4)
        def row_loop(l):
            lv = jnp.full((16,), 1, jnp.int32) * l
            v0 = rows_v[b, l, pl.ds(0, 16)]
            v1 = rows_v[b, l, pl.ds(16, 16)]
            plsc.store_scatter(block_v.at[b], [g0, s0, lv], v0)
            plsc.store_scatter(block_v.at[b], [g1, s0, lv], v1)

    for b in range(NBUF):
        fire_idx(b, b)
    for b in range(2):
        wait_idx(b)
        fire_gather(b)

    def group(g, carry):
        for b in range(NBUF):
            u = NBUF * g + b
            wait_gather(b)

            @pl.when(u + NBUF < U_PER_W)
            def _():
                fire_idx(u + NBUF, b)

            @pl.when(u >= NBUF)
            def _():
                wait_store(b)

            transpose(b)
            fire_store(u, b)

            b2 = (b + 2) % NBUF

            @pl.when(u + 2 < U_PER_W)
            def _():
                wait_idx(b2)
                fire_gather(b2)

        return carry

    lax.fori_loop(0, U_PER_W // NBUF, group, 0)
    for b in range(NBUF):
        wait_store(b)


@functools.partial(
    pl.kernel,
    mesh=plsc.VectorSubcoreMesh(core_axis_name="c", subcore_axis_name="s"),
    out_type=jax.ShapeDtypeStruct((B_COLS, 4, T_TILES, 8, 128), jnp.float32),
    scratch_types=[
        pltpu.VMEM((NBUF, 128), jnp.int32),
        pltpu.VMEM((NBUF, 128, EMB), jnp.float32),
        pltpu.VMEM((NBUF, 4, 8, 129), jnp.float32),
        (pltpu.SemaphoreType.DMA,) * NBUF,
        (pltpu.SemaphoreType.DMA,) * NBUF,
        (pltpu.SemaphoreType.DMA,) * NBUF,
    ],
    compiler_params=pltpu.CompilerParams(
        use_tc_tiling_on_sc=False, needs_layout_passes=False
    ),
)
def _emb_lookup(idx_hbm, tab_hbm, out_hbm, idx_v, rows_v, block_v, isems, gsems, ssems):
    _emb_body(idx_hbm, tab_hbm, out_hbm, idx_v, rows_v, block_v, (isems, gsems, ssems))


def kernel(vocab_ids, node_embs_weight):
    ids = vocab_ids.astype(jnp.int32)
    idx_native = ids.T.reshape(25, 8, T_TILES, 128).transpose(0, 2, 1, 3)
    out5 = _emb_lookup(idx_native, node_embs_weight)
    return out5.transpose(2, 4, 0, 1, 3).reshape(B_ROWS, B_COLS, EMB)


# speedup vs baseline: 2.0071x; 1.4715x over previous
import functools

import jax
import jax.numpy as jnp
from jax import lax
from jax.experimental import pallas as pl
from jax.experimental.pallas import tpu as pltpu
from jax.experimental.pallas import tpu_sc as plsc

EMB = 32
B_ROWS = 4096
B_COLS = 200
NW = 32
T_TILES = 32
UNITS = B_COLS * T_TILES
U_PER_W = UNITS // NW
NBUF = 4


def _emb_body(idx_hbm, tab_hbm, out_hbm, idx_v, rows_v, block_v, sems):
    isems, gsems, ssems = sems
    nc = 2
    wid = lax.axis_index("s") * nc + lax.axis_index("c")
    u_base = wid * U_PER_W

    l_iota = lax.iota(jnp.int32, 16)

    def unit_jt(u):
        uu = u_base + u
        j = uu // T_TILES
        t = uu % T_TILES
        return j, t

    def fire_idx(u, b):
        j, t = unit_jt(u)
        pltpu.async_copy(idx_hbm.at[j // 8, t, j % 8], idx_v.at[b], isems[b])

    def wait_idx(b):
        pltpu.make_async_copy(idx_hbm.at[0, 0, 0], idx_v.at[b], isems[b]).wait()

    def fire_gather(b):
        pltpu.async_copy(tab_hbm.at[idx_v.at[b]], rows_v.at[b], gsems[b])

    def wait_gather(b):
        pltpu.make_async_copy(tab_hbm.at[idx_v.at[b]], rows_v.at[b], gsems[b]).wait()

    def fire_store(u, b):
        j, t = unit_jt(u)
        pltpu.async_copy(block_v.at[b, :, :, pl.ds(0, 128)], out_hbm.at[j, :, t], ssems[b])

    def wait_store(b):
        pltpu.make_async_copy(
            block_v.at[b, :, :, pl.ds(0, 128)], out_hbm.at[0, :, 0], ssems[b]
        ).wait()

    g0 = l_iota // 8
    s0 = l_iota % 8
    g1 = g0 + 2

    def transpose(b):
        @plsc.parallel_loop(0, 128, 1, unroll=4)
        def row_loop(l):
            lv = jnp.full((16,), 1, jnp.int32) * l
            v0 = rows_v[b, l, pl.ds(0, 16)]
            v1 = rows_v[b, l, pl.ds(16, 16)]
            plsc.store_scatter(block_v.at[b], [g0, s0, lv], v0)
            plsc.store_scatter(block_v.at[b], [g1, s0, lv], v1)

    for b in range(NBUF):
        fire_idx(b, b)
    for b in range(2):
        wait_idx(b)
        fire_gather(b)

    def group(g, carry):
        for b in range(NBUF):
            u = NBUF * g + b
            wait_gather(b)

            @pl.when(u + NBUF < U_PER_W)
            def _():
                fire_idx(u + NBUF, b)

            @pl.when(u >= NBUF)
            def _():
                wait_store(b)

            transpose(b)
            fire_store(u, b)

            b2 = (b + 2) % NBUF

            @pl.when(u + 2 < U_PER_W)
            def _():
                wait_idx(b2)
                fire_gather(b2)

        return carry

    lax.fori_loop(0, U_PER_W // NBUF, group, 0)
    for b in range(NBUF):
        wait_store(b)


VPAD = 1000064
TT = VPAD // 128
TPW = -(-TT // NW)
NBUF2 = 4


def _tab_body(xt_hbm, tab_hbm, blk_v, rowsp_v, isems, ssems, l_iota):
    nc = 2
    wid = lax.axis_index("s") * nc + lax.axis_index("c")

    def fire_stage(n, b):
        t = wid + NW * n
        pltpu.async_copy(xt_hbm.at[:, t], blk_v.at[b], isems[b])

    def wait_stage(b):
        pltpu.make_async_copy(xt_hbm.at[:, 0], blk_v.at[b], isems[b]).wait()

    def fire_store(n, b):
        t = wid + NW * n
        pltpu.async_copy(
            rowsp_v.at[b, :, pl.ds(0, EMB)], tab_hbm.at[pl.ds(t * 128, 128)], ssems[b]
        )

    def wait_store(b):
        pltpu.make_async_copy(
            rowsp_v.at[b, :, pl.ds(0, EMB)], tab_hbm.at[pl.ds(0, 128)], ssems[b]
        ).wait()

    def transpose(b):
        @plsc.parallel_loop(0, EMB, 1, unroll=4)
        def kloop(k):
            kv = jnp.full((16,), 1, jnp.int32) * k
            for l0 in range(8):
                v = blk_v[b, k // 8, k % 8, pl.ds(16 * l0, 16)]
                plsc.store_scatter(rowsp_v.at[b], [l_iota + 16 * l0, kv], v)

    def in_range(n):
        return wid + NW * n < TT

    for b in range(NBUF2):
        @pl.when(in_range(b))
        def _():
            fire_stage(b, b)

    def group(gn, carry):
        for b in range(NBUF2):
            n = NBUF2 * gn + b

            @pl.when(in_range(n))
            def _():
                wait_stage(b)

                @pl.when(n >= NBUF2)
                def _():
                    wait_store(b)

                transpose(b)
                fire_store(n, b)

            @pl.when(in_range(n + NBUF2))
            def _():
                fire_stage(n + NBUF2, b)

        return carry

    lax.fori_loop(0, TPW // NBUF2 + 1, group, 0)
    for b in range(NBUF2):
        wait_store(b)


@functools.partial(
    pl.kernel,
    mesh=plsc.VectorSubcoreMesh(core_axis_name="c", subcore_axis_name="s"),
    out_type=jax.ShapeDtypeStruct((VPAD, EMB), jnp.float32),
    scratch_types=[
        pltpu.VMEM((NBUF2, 4, 8, 128), jnp.float32),
        pltpu.VMEM((NBUF2, 128, 33), jnp.float32),
        (pltpu.SemaphoreType.DMA,) * NBUF2,
        (pltpu.SemaphoreType.DMA,) * NBUF2,
    ],
    compiler_params=pltpu.CompilerParams(
        use_tc_tiling_on_sc=False, needs_layout_passes=False
    ),
)
def _tab_transpose(xt_hbm, tab_hbm, blk_v, rowsp_v, isems, ssems):
    _tab_body(xt_hbm, tab_hbm, blk_v, rowsp_v, isems, ssems, lax.iota(jnp.int32, 16))


@functools.partial(
    pl.kernel,
    mesh=plsc.VectorSubcoreMesh(core_axis_name="c", subcore_axis_name="s"),
    out_type=jax.ShapeDtypeStruct((B_COLS, 4, T_TILES, 8, 128), jnp.float32),
    scratch_types=[
        pltpu.VMEM((NBUF, 128), jnp.int32),
        pltpu.VMEM((NBUF, 128, EMB), jnp.float32),
        pltpu.VMEM((NBUF, 4, 8, 129), jnp.float32),
        (pltpu.SemaphoreType.DMA,) * NBUF,
        (pltpu.SemaphoreType.DMA,) * NBUF,
        (pltpu.SemaphoreType.DMA,) * NBUF,
    ],
    compiler_params=pltpu.CompilerParams(
        use_tc_tiling_on_sc=False, needs_layout_passes=False
    ),
)
def _emb_lookup(idx_hbm, tab_hbm, out_hbm, idx_v, rows_v, block_v, isems, gsems, ssems):
    _emb_body(idx_hbm, tab_hbm, out_hbm, idx_v, rows_v, block_v, (isems, gsems, ssems))


def kernel(vocab_ids, node_embs_weight):
    ids = vocab_ids.astype(jnp.int32)
    idx_native = ids.T.reshape(25, 8, T_TILES, 128).transpose(0, 2, 1, 3)
    tabx = jnp.pad(node_embs_weight, ((0, VPAD - 1000000), (0, 0)))
    xt = tabx.T.reshape(4, 8, TT, 128).transpose(0, 2, 1, 3)
    tab_lin = _tab_transpose(xt)
    out5 = _emb_lookup(idx_native, tab_lin)
    return out5.transpose(2, 4, 0, 1, 3).reshape(B_ROWS, B_COLS, EMB)
